# baseline (speedup 1.0000x reference)
import jax
import jax.numpy as jnp
from jax.experimental import pallas as pl
from jax.experimental.pallas import tpu as pltpu

_NUM = 256
_NPATCH = 256
_QDIM = 256
_KDIM = 256
_FB = 8
_INDIM = 48
_IMG = 64
_SCALE = 1.0 / (48.0 ** 0.5)


def _stage_a(rp_ref, wqt_ref, bq_ref, wkt_ref, bk_ref, svec_ref, bkq_ref,
             mcolt_ref, pa_ref, cm_ref):
    rp = rp_ref[0]
    c = jnp.dot(wqt_ref[...], rp, preferred_element_type=jnp.float32)
    bmat = (jnp.dot(c, wkt_ref[...], preferred_element_type=jnp.float32)
            + svec_ref[...] * bk_ref[...])
    u = jnp.dot(bq_ref[...], rp, preferred_element_type=jnp.float32)
    bqk = jnp.dot(u, wkt_ref[...], preferred_element_type=jnp.float32) + bkq_ref[...]
    att = jnp.dot(rp, bmat, preferred_element_type=jnp.float32) + bqk
    att = att - jnp.max(att, axis=1, keepdims=True)
    e = jnp.exp(att)
    e = e * (1.0 / jnp.sum(e, axis=1, keepdims=True))
    pa_ref[0, 0, :] = jnp.sum(e, axis=0)
    cm_ref[:, 0, 0, :] = jax.lax.dot_general(
        mcolt_ref[...], rp, (((1,), (1,)), ((), ())),
        preferred_element_type=jnp.float32)


def _stage_b(pa_ref, cm_ref, w1t_ref, b1_ref, w2t_ref, b2_ref,
             act_ref, sel_ref):
    pa = pa_ref[:, 0, :]
    iota = jax.lax.broadcasted_iota(jnp.int32, (_NUM, _NPATCH), 1)
    cols = []
    for _ in range(_FB):
        m = jnp.max(pa, axis=1, keepdims=True)
        eq = pa >= m
        idx = jnp.min(jnp.where(eq, iota, _NPATCH), axis=1, keepdims=True)
        sel = iota == idx
        pa = jnp.where(sel, -1.0, pa)
        row = idx // _IMG
        col = idx - row * _IMG
        cx = (row.astype(jnp.float32) + 2.0) * (1.0 / _IMG)
        cy = (col.astype(jnp.float32) + 2.0) * (1.0 / _IMG)
        r = jnp.sum(jnp.where(sel, cm_ref[0, :, 0, :], 0.0), axis=1, keepdims=True)
        g = jnp.sum(jnp.where(sel, cm_ref[1, :, 0, :], 0.0), axis=1, keepdims=True)
        b = jnp.sum(jnp.where(sel, cm_ref[2, :, 0, :], 0.0), axis=1, keepdims=True)
        cols.extend([cx, cy, r, g, b])
    feats = jnp.concatenate(cols, axis=1)
    h = jnp.dot(feats, w1t_ref[...], preferred_element_type=jnp.float32) + b1_ref[...]
    logits = jnp.dot(h, w2t_ref[...], preferred_element_type=jnp.float32) + b2_ref[...]
    lm = jnp.max(logits, axis=1, keepdims=True)
    e = jnp.exp(logits - lm)
    act_ref[...] = e / jnp.sum(e, axis=1, keepdims=True)
    li = jax.lax.broadcasted_iota(jnp.int32, logits.shape, 1)
    sel_idx = jnp.min(jnp.where(logits >= lm, li, logits.shape[1]), axis=1)
    sel_ref[0, :] = sel_idx


def kernel(input, Wq, bq, Wk, bk, W1, b1, W2, b2):
    rp = input.reshape(_NUM, _NPATCH, _INDIM)
    mcolt = jnp.zeros((8, _INDIM), jnp.float32)
    pix = jnp.arange(16)
    for c in range(3):
        mcolt = mcolt.at[c, pix * 3 + c].set(1.0 / (16.0 * 255.0))

    wqt_s = Wq.T * _SCALE
    bq_s = (bq * _SCALE).reshape(1, -1)
    wkt = Wk.T
    svec = jnp.sum(wqt_s, axis=1, keepdims=True)
    bkq = (jnp.sum(bq_s) * bk).reshape(1, -1)

    pa, cm = pl.pallas_call(
        _stage_a,
        grid=(_NUM,),
        in_specs=[
            pl.BlockSpec((1, _NPATCH, _INDIM), lambda b: (b, 0, 0)),
            pl.BlockSpec((_INDIM, _QDIM), lambda b: (0, 0)),
            pl.BlockSpec((1, _QDIM), lambda b: (0, 0)),
            pl.BlockSpec((_INDIM, _KDIM), lambda b: (0, 0)),
            pl.BlockSpec((1, _KDIM), lambda b: (0, 0)),
            pl.BlockSpec((_INDIM, 1), lambda b: (0, 0)),
            pl.BlockSpec((1, _KDIM), lambda b: (0, 0)),
            pl.BlockSpec((8, _INDIM), lambda b: (0, 0)),
        ],
        out_specs=[
            pl.BlockSpec((1, 1, _NPATCH), lambda b: (b, 0, 0)),
            pl.BlockSpec((8, 1, 1, _NPATCH), lambda b: (0, b, 0, 0)),
        ],
        out_shape=[
            jax.ShapeDtypeStruct((_NUM, 1, _NPATCH), jnp.float32),
            jax.ShapeDtypeStruct((8, _NUM, 1, _NPATCH), jnp.float32),
        ],
    )(rp, wqt_s, bq_s, wkt, bk.reshape(1, -1), svec, bkq, mcolt)

    actions, selected = pl.pallas_call(
        _stage_b,
        out_shape=[
            jax.ShapeDtypeStruct((_NUM, 15), jnp.float32),
            jax.ShapeDtypeStruct((1, _NUM), jnp.int32),
        ],
    )(pa, cm, W1.T, b1.reshape(1, -1), W2.T, b2.reshape(1, -1))

    return selected.reshape(_NUM), actions


# speedup vs baseline: 1.5668x; 1.5668x over previous
import jax
import jax.numpy as jnp
from jax.experimental import pallas as pl
from jax.experimental.pallas import tpu as pltpu

_NUM = 256
_NPATCH = 256
_QDIM = 256
_KDIM = 256
_FB = 8
_INDIM = 48
_IMG = 64
_SCALE = 1.0 / (48.0 ** 0.5)
_BB = 4


def _stage_a(rp_ref, wqt_ref, bq_ref, wkt_ref, bk_ref, mcolt_ref,
             pa_ref, cm_ref):
    for i in range(_BB):
        rp = rp_ref[i]
        q = jnp.dot(rp, wqt_ref[...], preferred_element_type=jnp.float32) + bq_ref[...]
        k = jnp.dot(rp, wkt_ref[...], preferred_element_type=jnp.float32) + bk_ref[...]
        att = jnp.dot(q, k, preferred_element_type=jnp.float32) * _SCALE
        att = att - jnp.max(att, axis=1, keepdims=True)
        e = jnp.exp(att)
        e = e * (1.0 / jnp.sum(e, axis=1, keepdims=True))
        pa_ref[i, 0, :] = jnp.sum(e, axis=0)
        cm_ref[:, i, 0, :] = jax.lax.dot_general(
            mcolt_ref[...], rp, (((1,), (1,)), ((), ())),
            preferred_element_type=jnp.float32)


def _stage_b(pa_ref, cm_ref, w1t_ref, b1_ref, w2t_ref, b2_ref,
             act_ref, sel_ref):
    pa = pa_ref[:, 0, :]
    iota = jax.lax.broadcasted_iota(jnp.int32, (_NUM, _NPATCH), 1)
    cols = []
    for _ in range(_FB):
        m = jnp.max(pa, axis=1, keepdims=True)
        eq = pa >= m
        idx = jnp.min(jnp.where(eq, iota, _NPATCH), axis=1, keepdims=True)
        sel = iota == idx
        pa = jnp.where(sel, -1.0, pa)
        row = idx // _IMG
        col = idx - row * _IMG
        cx = (row.astype(jnp.float32) + 2.0) * (1.0 / _IMG)
        cy = (col.astype(jnp.float32) + 2.0) * (1.0 / _IMG)
        r = jnp.sum(jnp.where(sel, cm_ref[0, :, 0, :], 0.0), axis=1, keepdims=True)
        g = jnp.sum(jnp.where(sel, cm_ref[1, :, 0, :], 0.0), axis=1, keepdims=True)
        b = jnp.sum(jnp.where(sel, cm_ref[2, :, 0, :], 0.0), axis=1, keepdims=True)
        cols.extend([cx, cy, r, g, b])
    feats = jnp.concatenate(cols, axis=1)
    h = jnp.dot(feats, w1t_ref[...], preferred_element_type=jnp.float32) + b1_ref[...]
    logits = jnp.dot(h, w2t_ref[...], preferred_element_type=jnp.float32) + b2_ref[...]
    lm = jnp.max(logits, axis=1, keepdims=True)
    e = jnp.exp(logits - lm)
    act_ref[...] = e / jnp.sum(e, axis=1, keepdims=True)
    li = jax.lax.broadcasted_iota(jnp.int32, logits.shape, 1)
    sel_idx = jnp.min(jnp.where(logits >= lm, li, logits.shape[1]), axis=1)
    sel_ref[0, :] = sel_idx


def kernel(input, Wq, bq, Wk, bk, W1, b1, W2, b2):
    rp = input.reshape(_NUM, _NPATCH, _INDIM)
    mcolt = jnp.zeros((8, _INDIM), jnp.float32)
    pix = jnp.arange(16)
    for c in range(3):
        mcolt = mcolt.at[c, pix * 3 + c].set(1.0 / (16.0 * 255.0))

    pa, cm = pl.pallas_call(
        _stage_a,
        grid=(_NUM // _BB,),
        in_specs=[
            pl.BlockSpec((_BB, _NPATCH, _INDIM), lambda b: (b, 0, 0)),
            pl.BlockSpec((_INDIM, _QDIM), lambda b: (0, 0)),
            pl.BlockSpec((1, _QDIM), lambda b: (0, 0)),
            pl.BlockSpec((_INDIM, _KDIM), lambda b: (0, 0)),
            pl.BlockSpec((1, _KDIM), lambda b: (0, 0)),
            pl.BlockSpec((8, _INDIM), lambda b: (0, 0)),
        ],
        out_specs=[
            pl.BlockSpec((_BB, 1, _NPATCH), lambda b: (b, 0, 0)),
            pl.BlockSpec((8, _BB, 1, _NPATCH), lambda b: (0, b, 0, 0)),
        ],
        out_shape=[
            jax.ShapeDtypeStruct((_NUM, 1, _NPATCH), jnp.float32),
            jax.ShapeDtypeStruct((8, _NUM, 1, _NPATCH), jnp.float32),
        ],
    )(rp, Wq.T, bq.reshape(1, -1), Wk.T, bk.reshape(1, -1), mcolt)

    actions, selected = pl.pallas_call(
        _stage_b,
        out_shape=[
            jax.ShapeDtypeStruct((_NUM, 15), jnp.float32),
            jax.ShapeDtypeStruct((1, _NUM), jnp.int32),
        ],
    )(pa, cm, W1.T, b1.reshape(1, -1), W2.T, b2.reshape(1, -1))

    return selected.reshape(_NUM), actions


# speedup vs baseline: 1.6712x; 1.0666x over previous
import jax
import jax.numpy as jnp
from jax.experimental import pallas as pl
from jax.experimental.pallas import tpu as pltpu

_NUM = 256
_NPATCH = 256
_QDIM = 256
_KDIM = 256
_FB = 8
_INDIM = 48
_IMG = 64
_SCALE = 1.0 / (48.0 ** 0.5)
_BB = 8


def _stage_a(rp_ref, wqt_ref, bq_ref, wkt_ref, bk_ref, mcolt_ref,
             pa_ref, cm_ref):
    for i in range(_BB):
        rp = rp_ref[i]
        q = jnp.dot(rp, wqt_ref[...], preferred_element_type=jnp.float32) + bq_ref[...]
        k = jnp.dot(rp, wkt_ref[...], preferred_element_type=jnp.float32) + bk_ref[...]
        att = jnp.dot(q, k, preferred_element_type=jnp.float32) * _SCALE
        att = att - jnp.max(att, axis=1, keepdims=True)
        e = jnp.exp(att)
        e = e * (1.0 / jnp.sum(e, axis=1, keepdims=True))
        pa_ref[i, 0, :] = jnp.sum(e, axis=0)
        cm_ref[:, i, 0, :] = jax.lax.dot_general(
            mcolt_ref[...], rp, (((1,), (1,)), ((), ())),
            preferred_element_type=jnp.float32)


def _stage_b(pa_ref, cm_ref, w1t_ref, b1_ref, w2t_ref, b2_ref,
             act_ref, sel_ref):
    pa = pa_ref[:, 0, :]
    iota = jax.lax.broadcasted_iota(jnp.int32, (_NUM, _NPATCH), 1)
    cols = []
    for _ in range(_FB):
        m = jnp.max(pa, axis=1, keepdims=True)
        eq = pa >= m
        idx = jnp.min(jnp.where(eq, iota, _NPATCH), axis=1, keepdims=True)
        sel = iota == idx
        pa = jnp.where(sel, -1.0, pa)
        row = idx // _IMG
        col = idx - row * _IMG
        cx = (row.astype(jnp.float32) + 2.0) * (1.0 / _IMG)
        cy = (col.astype(jnp.float32) + 2.0) * (1.0 / _IMG)
        r = jnp.sum(jnp.where(sel, cm_ref[0, :, 0, :], 0.0), axis=1, keepdims=True)
        g = jnp.sum(jnp.where(sel, cm_ref[1, :, 0, :], 0.0), axis=1, keepdims=True)
        b = jnp.sum(jnp.where(sel, cm_ref[2, :, 0, :], 0.0), axis=1, keepdims=True)
        cols.extend([cx, cy, r, g, b])
    feats = jnp.concatenate(cols, axis=1)
    h = jnp.dot(feats, w1t_ref[...], preferred_element_type=jnp.float32) + b1_ref[...]
    logits = jnp.dot(h, w2t_ref[...], preferred_element_type=jnp.float32) + b2_ref[...]
    lm = jnp.max(logits, axis=1, keepdims=True)
    e = jnp.exp(logits - lm)
    act_ref[...] = e / jnp.sum(e, axis=1, keepdims=True)
    li = jax.lax.broadcasted_iota(jnp.int32, logits.shape, 1)
    sel_idx = jnp.min(jnp.where(logits >= lm, li, logits.shape[1]), axis=1)
    sel_ref[0, :] = sel_idx


def kernel(input, Wq, bq, Wk, bk, W1, b1, W2, b2):
    rp = input.reshape(_NUM, _NPATCH, _INDIM)
    mcolt = jnp.zeros((8, _INDIM), jnp.float32)
    pix = jnp.arange(16)
    for c in range(3):
        mcolt = mcolt.at[c, pix * 3 + c].set(1.0 / (16.0 * 255.0))

    pa, cm = pl.pallas_call(
        _stage_a,
        grid=(_NUM // _BB,),
        in_specs=[
            pl.BlockSpec((_BB, _NPATCH, _INDIM), lambda b: (b, 0, 0)),
            pl.BlockSpec((_INDIM, _QDIM), lambda b: (0, 0)),
            pl.BlockSpec((1, _QDIM), lambda b: (0, 0)),
            pl.BlockSpec((_INDIM, _KDIM), lambda b: (0, 0)),
            pl.BlockSpec((1, _KDIM), lambda b: (0, 0)),
            pl.BlockSpec((8, _INDIM), lambda b: (0, 0)),
        ],
        out_specs=[
            pl.BlockSpec((_BB, 1, _NPATCH), lambda b: (b, 0, 0)),
            pl.BlockSpec((8, _BB, 1, _NPATCH), lambda b: (0, b, 0, 0)),
        ],
        out_shape=[
            jax.ShapeDtypeStruct((_NUM, 1, _NPATCH), jnp.float32),
            jax.ShapeDtypeStruct((8, _NUM, 1, _NPATCH), jnp.float32),
        ],
    )(rp, Wq.T, bq.reshape(1, -1), Wk.T, bk.reshape(1, -1), mcolt)

    actions, selected = pl.pallas_call(
        _stage_b,
        out_shape=[
            jax.ShapeDtypeStruct((_NUM, 15), jnp.float32),
            jax.ShapeDtypeStruct((1, _NUM), jnp.int32),
        ],
    )(pa, cm, W1.T, b1.reshape(1, -1), W2.T, b2.reshape(1, -1))

    return selected.reshape(_NUM), actions
